# baseline (speedup 1.0000x reference)
import functools

import jax
import jax.numpy as jnp
from jax.experimental import pallas as pl
from jax.experimental.pallas import tpu as pltpu

N = 2048
F_IN = 2048
HID = 64
HEADS = 8


def _elu(v):
    return jnp.where(v > 0, v, jnp.exp(v) - 1.0)


def _lrelu(v):
    return jnp.maximum(v, 0.2 * v)


def _proj_kernel(x_ref, w_ref, asd_ref, h_ref, s_ref, d_ref, *, tdim):
    h = jax.lax.dot_general(
        x_ref[...].astype(jnp.bfloat16), w_ref[...].astype(jnp.bfloat16),
        (((tdim,), (0,)), ((), ())),
        preferred_element_type=jnp.float32)
    h_ref[...] = h.astype(jnp.bfloat16)
    sd = jax.lax.dot_general(
        h, asd_ref[...], (((1,), (0,)), ((), ())),
        preferred_element_type=jnp.float32)
    nh = asd_ref.shape[1] // 2
    s_ref[...] = sd[:, :nh]
    d_ref[...] = sd[:, nh:]


def _attn_kernel(s_ref, d_ref, h_ref, b_ref, out_ref, *, heads):
    for hd in range(heads):
        s_row = s_ref[hd:hd + 1, :]
        u1 = jnp.exp2(s_row).astype(jnp.bfloat16)
        u2 = jnp.exp2(0.2 * s_row).astype(jnp.bfloat16)
        smax = jnp.max(s_row)
        d_col = d_ref[:, hd:hd + 1]
        m = _lrelu(d_col + smax)
        v1 = jnp.exp2(d_col - m).astype(jnp.bfloat16)
        v2 = jnp.exp2(0.2 * d_col - m).astype(jnp.bfloat16)
        p = jnp.maximum(v1 * u1, v2 * u2)
        h_ext = jnp.concatenate(
            [h_ref[:, hd * HID:(hd + 1) * HID],
             jnp.ones((N, 1), jnp.bfloat16)], axis=1)
        pe = jax.lax.dot_general(
            p, h_ext, (((1,), (0,)), ((), ())),
            preferred_element_type=jnp.float32)
        out_ref[:, hd * HID:(hd + 1) * HID] = _elu(
            pe[:, :HID] / pe[:, HID:HID + 1]
            + b_ref[0:1, hd * HID:(hd + 1) * HID]).astype(out_ref.dtype)


def _projection(xin, w, asd, tdim, bm):
    nprog = N // bm
    kdim = w.shape[0]
    fdim = w.shape[1]
    nh = asd.shape[1] // 2
    if tdim == 0:
        x_spec = pl.BlockSpec((kdim, bm), lambda j: (0, j))
    else:
        x_spec = pl.BlockSpec((bm, kdim), lambda j: (j, 0))
    return pl.pallas_call(
        functools.partial(_proj_kernel, tdim=tdim),
        grid=(nprog,),
        in_specs=[
            x_spec,
            pl.BlockSpec((kdim, fdim), lambda j: (0, 0)),
            pl.BlockSpec((fdim, 2 * nh), lambda j: (0, 0)),
        ],
        out_specs=[
            pl.BlockSpec((bm, fdim), lambda j: (j, 0)),
            pl.BlockSpec((bm, nh), lambda j: (j, 0)),
            pl.BlockSpec((bm, nh), lambda j: (j, 0)),
        ],
        out_shape=[
            jax.ShapeDtypeStruct((N, fdim), jnp.bfloat16),
            jax.ShapeDtypeStruct((N, nh), jnp.float32),
            jax.ShapeDtypeStruct((N, nh), jnp.float32),
        ],
    )(xin, w, asd)


def _attn1_proj_kernel(s_ref, d_ref, h_ref, b_ref, w2_ref, asd2_ref,
                       h2_ref, s2_ref, d2_ref, *, heads):
    outs = []
    for hd in range(heads):
        s_row = s_ref[hd:hd + 1, :]
        u1 = jnp.exp2(s_row).astype(jnp.bfloat16)
        u2 = jnp.exp2(0.2 * s_row).astype(jnp.bfloat16)
        smax = jnp.max(s_row)
        d_col = d_ref[:, hd:hd + 1]
        m = _lrelu(d_col + smax)
        v1 = jnp.exp2(d_col - m).astype(jnp.bfloat16)
        v2 = jnp.exp2(0.2 * d_col - m).astype(jnp.bfloat16)
        p = jnp.maximum(v1 * u1, v2 * u2)
        h_ext = jnp.concatenate(
            [h_ref[:, hd * HID:(hd + 1) * HID],
             jnp.ones((N, 1), jnp.bfloat16)], axis=1)
        pe = jax.lax.dot_general(
            p, h_ext, (((1,), (0,)), ((), ())),
            preferred_element_type=jnp.float32)
        outs.append(_elu(
            pe[:, :HID] / pe[:, HID:HID + 1]
            + b_ref[0:1, hd * HID:(hd + 1) * HID]).astype(jnp.bfloat16))
    out1 = jnp.concatenate(outs, axis=1)
    h2 = jax.lax.dot_general(
        out1, w2_ref[...].astype(jnp.bfloat16), (((1,), (0,)), ((), ())),
        preferred_element_type=jnp.float32)
    h2_ref[...] = h2.astype(jnp.bfloat16)
    sd = jax.lax.dot_general(
        h2, asd2_ref[...], (((1,), (0,)), ((), ())),
        preferred_element_type=jnp.float32)
    s2_ref[...] = sd[:, 0:1]
    d2_ref[...] = sd[:, 1:2]


def _attn1_proj(s_r, d_t, h, b_r, w2, asd2, bi):
    ni = N // bi
    fdim = HEADS * HID
    return pl.pallas_call(
        functools.partial(_attn1_proj_kernel, heads=HEADS),
        grid=(ni,),
        in_specs=[
            pl.BlockSpec((HEADS, N), lambda it: (0, 0)),
            pl.BlockSpec((bi, HEADS), lambda it: (it, 0)),
            pl.BlockSpec((N, fdim), lambda it: (0, 0)),
            pl.BlockSpec((1, fdim), lambda it: (0, 0)),
            pl.BlockSpec((fdim, HID), lambda it: (0, 0)),
            pl.BlockSpec((HID, 2), lambda it: (0, 0)),
        ],
        out_specs=[
            pl.BlockSpec((bi, HID), lambda it: (it, 0)),
            pl.BlockSpec((bi, 1), lambda it: (it, 0)),
            pl.BlockSpec((bi, 1), lambda it: (it, 0)),
        ],
        out_shape=[
            jax.ShapeDtypeStruct((N, HID), jnp.bfloat16),
            jax.ShapeDtypeStruct((N, 1), jnp.float32),
            jax.ShapeDtypeStruct((N, 1), jnp.float32),
        ],
        compiler_params=pltpu.CompilerParams(
            dimension_semantics=("parallel",)),
    )(s_r, d_t, h, b_r, w2, asd2)


def _attention(s_r, d_t, h, b_r, heads, bi, out_dtype):
    ni = N // bi
    fdim = heads * HID
    return pl.pallas_call(
        functools.partial(_attn_kernel, heads=heads),
        grid=(ni,),
        in_specs=[
            pl.BlockSpec((heads, N), lambda it: (0, 0)),
            pl.BlockSpec((bi, heads), lambda it: (it, 0)),
            pl.BlockSpec((N, fdim), lambda it: (0, 0)),
            pl.BlockSpec((1, fdim), lambda it: (0, 0)),
        ],
        out_specs=pl.BlockSpec((bi, fdim), lambda it: (it, 0)),
        out_shape=jax.ShapeDtypeStruct((N, fdim), out_dtype),
        compiler_params=pltpu.CompilerParams(
            dimension_semantics=("parallel",)),
    )(s_r, d_t, h, b_r)


def kernel(x, W1, a1_src, a1_dst, b1, W2, a2_src, a2_dst, b2):
    log2e = jnp.float32(1.4426950408889634)
    eye1 = jnp.eye(HEADS, dtype=jnp.float32)
    A1s = (eye1[:, None, :] * a1_src[:, :, None]).reshape(HEADS * HID, HEADS)
    A1d = (eye1[:, None, :] * a1_dst[:, :, None]).reshape(HEADS * HID, HEADS)
    asd1 = jnp.concatenate([A1s, A1d], axis=1) * log2e
    asd2 = jnp.concatenate([a2_src.T, a2_dst.T], axis=1) * log2e

    h1, s1, d1 = _projection(x, W1, asd1, tdim=0, bm=256)
    h2, s2, d2 = _attn1_proj(s1.T, d1, h1, b1.reshape(1, HEADS * HID),
                             W2, asd2, bi=2048)
    out2 = _attention(s2.T, d2, h2, b2.reshape(1, HID), 1,
                      bi=512, out_dtype=jnp.float32)
    return out2.reshape(1, N * HID)


# speedup vs baseline: 1.0737x; 1.0737x over previous
import functools

import jax
import jax.numpy as jnp
from jax.experimental import pallas as pl
from jax.experimental.pallas import tpu as pltpu

N = 2048
F_IN = 2048
HID = 64
HEADS = 8


def _elu(v):
    return jnp.where(v > 0, v, jnp.exp(v) - 1.0)


def _lrelu(v):
    return jnp.maximum(v, 0.2 * v)


def _proj_kernel(x_ref, w_ref, asd_ref, h_ref, s_ref, d_ref, *, tdim):
    h = jax.lax.dot_general(
        x_ref[...].astype(jnp.bfloat16), w_ref[...].astype(jnp.bfloat16),
        (((tdim,), (0,)), ((), ())),
        preferred_element_type=jnp.float32)
    h_ref[...] = h.astype(jnp.bfloat16)
    sd = jax.lax.dot_general(
        h, asd_ref[...], (((1,), (0,)), ((), ())),
        preferred_element_type=jnp.float32)
    nh = asd_ref.shape[1] // 2
    s_ref[...] = sd[:, :nh]
    d_ref[...] = sd[:, nh:]


def _attn_kernel(s_ref, d_ref, h_ref, b_ref, out_ref, *, heads):
    for hd in range(heads):
        s_row = s_ref[hd:hd + 1, :]
        u1 = jnp.exp2(s_row).astype(jnp.bfloat16)
        u2 = jnp.exp2(0.2 * s_row).astype(jnp.bfloat16)
        smax = jnp.max(s_row)
        d_col = d_ref[:, hd:hd + 1]
        m = _lrelu(d_col + smax)
        v1 = jnp.exp2(d_col - m).astype(jnp.bfloat16)
        v2 = jnp.exp2(0.2 * d_col - m).astype(jnp.bfloat16)
        p = jnp.maximum(v1 * u1, v2 * u2)
        h_ext = jnp.concatenate(
            [h_ref[:, hd * HID:(hd + 1) * HID],
             jnp.ones((N, 1), jnp.bfloat16)], axis=1)
        pe = jax.lax.dot_general(
            p, h_ext, (((1,), (0,)), ((), ())),
            preferred_element_type=jnp.float32)
        out_ref[:, hd * HID:(hd + 1) * HID] = _elu(
            pe[:, :HID] / pe[:, HID:HID + 1]
            + b_ref[0:1, hd * HID:(hd + 1) * HID]).astype(out_ref.dtype)


def _projection(xin, w, asd, tdim, bm):
    nprog = N // bm
    kdim = w.shape[0]
    fdim = w.shape[1]
    nh = asd.shape[1] // 2
    if tdim == 0:
        x_spec = pl.BlockSpec((kdim, bm), lambda j: (0, j))
    else:
        x_spec = pl.BlockSpec((bm, kdim), lambda j: (j, 0))
    return pl.pallas_call(
        functools.partial(_proj_kernel, tdim=tdim),
        grid=(nprog,),
        in_specs=[
            x_spec,
            pl.BlockSpec((kdim, fdim), lambda j: (0, 0)),
            pl.BlockSpec((fdim, 2 * nh), lambda j: (0, 0)),
        ],
        out_specs=[
            pl.BlockSpec((bm, fdim), lambda j: (j, 0)),
            pl.BlockSpec((bm, nh), lambda j: (j, 0)),
            pl.BlockSpec((bm, nh), lambda j: (j, 0)),
        ],
        out_shape=[
            jax.ShapeDtypeStruct((N, fdim), jnp.bfloat16),
            jax.ShapeDtypeStruct((N, nh), jnp.float32),
            jax.ShapeDtypeStruct((N, nh), jnp.float32),
        ],
    )(xin, w, asd)


def _attn1_proj_kernel(s_ref, d_ref, h_ref, b_ref, w2_ref, asd2_ref,
                       h2_ref, s2_ref, d2_ref, *, heads):
    outs = []
    for hd in range(heads):
        s_row = s_ref[hd:hd + 1, :]
        u1 = jnp.exp2(s_row).astype(jnp.bfloat16)
        u2 = jnp.exp2(0.2 * s_row).astype(jnp.bfloat16)
        smax = jnp.max(s_row)
        d_col = d_ref[:, hd:hd + 1]
        m = _lrelu(d_col + smax)
        v1 = jnp.exp2(d_col - m).astype(jnp.bfloat16)
        v2 = jnp.exp2(0.2 * d_col - m).astype(jnp.bfloat16)
        p = jnp.maximum(v1 * u1, v2 * u2)
        h_ext = jnp.concatenate(
            [h_ref[:, hd * HID:(hd + 1) * HID],
             jnp.ones((N, 1), jnp.bfloat16)], axis=1)
        pe = jax.lax.dot_general(
            p, h_ext, (((1,), (0,)), ((), ())),
            preferred_element_type=jnp.float32)
        outs.append(_elu(
            pe[:, :HID] / pe[:, HID:HID + 1]
            + b_ref[0:1, hd * HID:(hd + 1) * HID]).astype(jnp.bfloat16))
    out1 = jnp.concatenate(outs, axis=1)
    h2 = jax.lax.dot_general(
        out1, w2_ref[...].astype(jnp.bfloat16), (((1,), (0,)), ((), ())),
        preferred_element_type=jnp.float32)
    h2_ref[...] = h2.astype(jnp.bfloat16)
    sd = jax.lax.dot_general(
        h2, asd2_ref[...], (((1,), (0,)), ((), ())),
        preferred_element_type=jnp.float32)
    s2_ref[...] = sd[:, 0:1]
    d2_ref[...] = sd[:, 1:2]


def _attn1_proj(s_r, d_t, h, b_r, w2, asd2, bi):
    ni = N // bi
    fdim = HEADS * HID
    return pl.pallas_call(
        functools.partial(_attn1_proj_kernel, heads=HEADS),
        grid=(ni,),
        in_specs=[
            pl.BlockSpec((HEADS, N), lambda it: (0, 0)),
            pl.BlockSpec((bi, HEADS), lambda it: (it, 0)),
            pl.BlockSpec((N, fdim), lambda it: (0, 0)),
            pl.BlockSpec((1, fdim), lambda it: (0, 0)),
            pl.BlockSpec((fdim, HID), lambda it: (0, 0)),
            pl.BlockSpec((HID, 2), lambda it: (0, 0)),
        ],
        out_specs=[
            pl.BlockSpec((bi, HID), lambda it: (it, 0)),
            pl.BlockSpec((bi, 1), lambda it: (it, 0)),
            pl.BlockSpec((bi, 1), lambda it: (it, 0)),
        ],
        out_shape=[
            jax.ShapeDtypeStruct((N, HID), jnp.bfloat16),
            jax.ShapeDtypeStruct((N, 1), jnp.float32),
            jax.ShapeDtypeStruct((N, 1), jnp.float32),
        ],
        compiler_params=pltpu.CompilerParams(
            dimension_semantics=("parallel",)),
    )(s_r, d_t, h, b_r, w2, asd2)


def _attention(s_r, d_t, h, b_r, heads, bi, out_dtype):
    ni = N // bi
    fdim = heads * HID
    return pl.pallas_call(
        functools.partial(_attn_kernel, heads=heads),
        grid=(ni,),
        in_specs=[
            pl.BlockSpec((heads, N), lambda it: (0, 0)),
            pl.BlockSpec((bi, heads), lambda it: (it, 0)),
            pl.BlockSpec((N, fdim), lambda it: (0, 0)),
            pl.BlockSpec((1, fdim), lambda it: (0, 0)),
        ],
        out_specs=pl.BlockSpec((bi, fdim), lambda it: (it, 0)),
        out_shape=jax.ShapeDtypeStruct((N, fdim), out_dtype),
        compiler_params=pltpu.CompilerParams(
            dimension_semantics=("parallel",)),
    )(s_r, d_t, h, b_r)


def kernel(x, W1, a1_src, a1_dst, b1, W2, a2_src, a2_dst, b2):
    log2e = jnp.float32(1.4426950408889634)
    eye1 = jnp.eye(HEADS, dtype=jnp.float32)
    A1s = (eye1[:, None, :] * a1_src[:, :, None]).reshape(HEADS * HID, HEADS)
    A1d = (eye1[:, None, :] * a1_dst[:, :, None]).reshape(HEADS * HID, HEADS)
    asd1 = jnp.concatenate([A1s, A1d], axis=1) * log2e
    asd2 = jnp.concatenate([a2_src.T, a2_dst.T], axis=1) * log2e

    h1, s1, d1 = _projection(x, W1, asd1, tdim=0, bm=512)
    h2, s2, d2 = _attn1_proj(s1.T, d1, h1, b1.reshape(1, HEADS * HID),
                             W2, asd2, bi=1024)
    out2 = _attention(s2.T, d2, h2, b2.reshape(1, HID), 1,
                      bi=1024, out_dtype=jnp.float32)
    return out2.reshape(1, N * HID)


# speedup vs baseline: 1.0778x; 1.0039x over previous
import functools

import jax
import jax.numpy as jnp
from jax.experimental import pallas as pl
from jax.experimental.pallas import tpu as pltpu

N = 2048
F_IN = 2048
HID = 64
HEADS = 8


def _elu(v):
    return jnp.where(v > 0, v, jnp.exp(v) - 1.0)


def _lrelu(v):
    return jnp.maximum(v, 0.2 * v)


def _proj_kernel(x_ref, w_ref, asd_ref, h_ref, s_ref, d_ref, *, tdim):
    h = jax.lax.dot_general(
        x_ref[...].astype(jnp.bfloat16), w_ref[...].astype(jnp.bfloat16),
        (((tdim,), (0,)), ((), ())),
        preferred_element_type=jnp.float32)
    h_ref[...] = h.astype(jnp.bfloat16)
    sd = jax.lax.dot_general(
        h, asd_ref[...], (((1,), (0,)), ((), ())),
        preferred_element_type=jnp.float32)
    nh = asd_ref.shape[1] // 2
    s_ref[...] = sd[:, :nh]
    d_ref[...] = sd[:, nh:]


def _attn_kernel(s_ref, d_ref, h_ref, b_ref, out_ref, *, heads):
    for hd in range(heads):
        s_row = s_ref[hd:hd + 1, :]
        u1 = jnp.exp2(s_row).astype(jnp.bfloat16)
        u2 = jnp.exp2(0.2 * s_row).astype(jnp.bfloat16)
        smax = jnp.max(s_row)
        d_col = d_ref[:, hd:hd + 1]
        m = _lrelu(d_col + smax)
        v1 = jnp.exp2(d_col - m).astype(jnp.bfloat16)
        v2 = jnp.exp2(0.2 * d_col - m).astype(jnp.bfloat16)
        p = jnp.maximum(v1 * u1, v2 * u2)
        h_ext = jnp.concatenate(
            [h_ref[:, hd * HID:(hd + 1) * HID],
             jnp.ones((N, 1), jnp.bfloat16)], axis=1)
        pe = jax.lax.dot_general(
            p, h_ext, (((1,), (0,)), ((), ())),
            preferred_element_type=jnp.float32)
        out_ref[:, hd * HID:(hd + 1) * HID] = _elu(
            pe[:, :HID] / pe[:, HID:HID + 1]
            + b_ref[0:1, hd * HID:(hd + 1) * HID]).astype(out_ref.dtype)


def _projection(xin, w, asd, tdim, bm):
    nprog = N // bm
    kdim = w.shape[0]
    fdim = w.shape[1]
    nh = asd.shape[1] // 2
    if tdim == 0:
        x_spec = pl.BlockSpec((kdim, bm), lambda j: (0, j))
    else:
        x_spec = pl.BlockSpec((bm, kdim), lambda j: (j, 0))
    return pl.pallas_call(
        functools.partial(_proj_kernel, tdim=tdim),
        grid=(nprog,),
        in_specs=[
            x_spec,
            pl.BlockSpec((kdim, fdim), lambda j: (0, 0)),
            pl.BlockSpec((fdim, 2 * nh), lambda j: (0, 0)),
        ],
        out_specs=[
            pl.BlockSpec((bm, fdim), lambda j: (j, 0)),
            pl.BlockSpec((bm, nh), lambda j: (j, 0)),
            pl.BlockSpec((bm, nh), lambda j: (j, 0)),
        ],
        out_shape=[
            jax.ShapeDtypeStruct((N, fdim), jnp.bfloat16),
            jax.ShapeDtypeStruct((N, nh), jnp.float32),
            jax.ShapeDtypeStruct((N, nh), jnp.float32),
        ],
    )(xin, w, asd)


def _attn1_proj_kernel(s_ref, d_ref, h_ref, b_ref, w2_ref, asd2_ref,
                       h2_ref, s2_ref, d2_ref, *, heads):
    outs = []
    for hd in range(heads):
        s_row = s_ref[hd:hd + 1, :]
        u1 = jnp.exp2(s_row).astype(jnp.bfloat16)
        u2 = jnp.exp2(0.2 * s_row).astype(jnp.bfloat16)
        smax = jnp.max(s_row)
        d_col = d_ref[:, hd:hd + 1]
        m = _lrelu(d_col + smax)
        v1 = jnp.exp2(d_col - m).astype(jnp.bfloat16)
        v2 = jnp.exp2(0.2 * d_col - m).astype(jnp.bfloat16)
        p = jnp.maximum(v1 * u1, v2 * u2)
        h_ext = jnp.concatenate(
            [h_ref[:, hd * HID:(hd + 1) * HID],
             jnp.ones((N, 1), jnp.bfloat16)], axis=1)
        pe = jax.lax.dot_general(
            p, h_ext, (((1,), (0,)), ((), ())),
            preferred_element_type=jnp.float32)
        outs.append(_elu(
            pe[:, :HID] / pe[:, HID:HID + 1]
            + b_ref[0:1, hd * HID:(hd + 1) * HID]).astype(jnp.bfloat16))
    out1 = jnp.concatenate(outs, axis=1)
    h2 = jax.lax.dot_general(
        out1, w2_ref[...].astype(jnp.bfloat16), (((1,), (0,)), ((), ())),
        preferred_element_type=jnp.float32)
    h2_ref[...] = h2.astype(jnp.bfloat16)
    sd = jax.lax.dot_general(
        h2, asd2_ref[...], (((1,), (0,)), ((), ())),
        preferred_element_type=jnp.float32)
    s2_ref[...] = sd[:, 0:1]
    d2_ref[...] = sd[:, 1:2]


def _attn1_proj(s_r, d_t, h, b_r, w2, asd2, bi):
    ni = N // bi
    fdim = HEADS * HID
    return pl.pallas_call(
        functools.partial(_attn1_proj_kernel, heads=HEADS),
        grid=(ni,),
        in_specs=[
            pl.BlockSpec((HEADS, N), lambda it: (0, 0)),
            pl.BlockSpec((bi, HEADS), lambda it: (it, 0)),
            pl.BlockSpec((N, fdim), lambda it: (0, 0)),
            pl.BlockSpec((1, fdim), lambda it: (0, 0)),
            pl.BlockSpec((fdim, HID), lambda it: (0, 0)),
            pl.BlockSpec((HID, 2), lambda it: (0, 0)),
        ],
        out_specs=[
            pl.BlockSpec((bi, HID), lambda it: (it, 0)),
            pl.BlockSpec((bi, 1), lambda it: (it, 0)),
            pl.BlockSpec((bi, 1), lambda it: (it, 0)),
        ],
        out_shape=[
            jax.ShapeDtypeStruct((N, HID), jnp.bfloat16),
            jax.ShapeDtypeStruct((N, 1), jnp.float32),
            jax.ShapeDtypeStruct((N, 1), jnp.float32),
        ],
        compiler_params=pltpu.CompilerParams(
            dimension_semantics=("parallel",)),
    )(s_r, d_t, h, b_r, w2, asd2)


def _attention(s_r, d_t, h, b_r, heads, bi, out_dtype):
    ni = N // bi
    fdim = heads * HID
    return pl.pallas_call(
        functools.partial(_attn_kernel, heads=heads),
        grid=(ni,),
        in_specs=[
            pl.BlockSpec((heads, N), lambda it: (0, 0)),
            pl.BlockSpec((bi, heads), lambda it: (it, 0)),
            pl.BlockSpec((N, fdim), lambda it: (0, 0)),
            pl.BlockSpec((1, fdim), lambda it: (0, 0)),
        ],
        out_specs=pl.BlockSpec((bi, fdim), lambda it: (it, 0)),
        out_shape=jax.ShapeDtypeStruct((N, fdim), out_dtype),
        compiler_params=pltpu.CompilerParams(
            dimension_semantics=("parallel",)),
    )(s_r, d_t, h, b_r)


def kernel(x, W1, a1_src, a1_dst, b1, W2, a2_src, a2_dst, b2):
    log2e = jnp.float32(1.4426950408889634)
    eye1 = jnp.eye(HEADS, dtype=jnp.float32)
    A1s = (eye1[:, None, :] * a1_src[:, :, None]).reshape(HEADS * HID, HEADS)
    A1d = (eye1[:, None, :] * a1_dst[:, :, None]).reshape(HEADS * HID, HEADS)
    asd1 = jnp.concatenate([A1s, A1d], axis=1) * log2e
    asd2 = jnp.concatenate([a2_src.T, a2_dst.T], axis=1) * log2e

    h1, s1, d1 = _projection(x, W1, asd1, tdim=0, bm=1024)
    h2, s2, d2 = _attn1_proj(s1.T, d1, h1, b1.reshape(1, HEADS * HID),
                             W2, asd2, bi=1024)
    out2 = _attention(s2.T, d2, h2, b2.reshape(1, HID), 1,
                      bi=1024, out_dtype=jnp.float32)
    return out2.reshape(1, N * HID)


# speedup vs baseline: 1.1775x; 1.0925x over previous
import functools

import jax
import jax.numpy as jnp
from jax.experimental import pallas as pl
from jax.experimental.pallas import tpu as pltpu

N = 2048
F_IN = 2048
HID = 64
HEADS = 8


def _elu(v):
    return jnp.where(v > 0, v, jnp.exp(v) - 1.0)


def _lrelu(v):
    return jnp.maximum(v, 0.2 * v)


def _proj_kernel(x_ref, w_ref, asd_ref, h_ref, s_ref, d_ref, *, tdim):
    h = jax.lax.dot_general(
        x_ref[...].astype(jnp.bfloat16), w_ref[...].astype(jnp.bfloat16),
        (((tdim,), (0,)), ((), ())),
        preferred_element_type=jnp.float32)
    h_ref[...] = h.astype(jnp.bfloat16)
    sd = jax.lax.dot_general(
        h, asd_ref[...], (((1,), (0,)), ((), ())),
        preferred_element_type=jnp.float32)
    nh = asd_ref.shape[1] // 2
    s_ref[...] = sd[:, :nh].T
    d_ref[...] = sd[:, nh:]


def _attn_kernel(s_ref, d_ref, h_ref, b_ref, out_ref, *, heads):
    for hd in range(heads):
        s_row = s_ref[hd:hd + 1, :]
        u1 = jnp.exp2(s_row).astype(jnp.bfloat16)
        u2 = jnp.exp2(0.2 * s_row).astype(jnp.bfloat16)
        smax = jnp.max(s_row)
        d_col = d_ref[:, hd:hd + 1]
        m = _lrelu(d_col + smax)
        v1 = jnp.exp2(d_col - m).astype(jnp.bfloat16)
        v2 = jnp.exp2(0.2 * d_col - m).astype(jnp.bfloat16)
        p = jnp.maximum(v1 * u1, v2 * u2)
        h_ext = jnp.concatenate(
            [h_ref[:, hd * HID:(hd + 1) * HID],
             jnp.ones((N, 1), jnp.bfloat16)], axis=1)
        pe = jax.lax.dot_general(
            p, h_ext, (((1,), (0,)), ((), ())),
            preferred_element_type=jnp.float32)
        out_ref[:, hd * HID:(hd + 1) * HID] = _elu(
            pe[:, :HID] / pe[:, HID:HID + 1]
            + b_ref[0:1, hd * HID:(hd + 1) * HID]).astype(out_ref.dtype)


def _projection(xin, w, asd, tdim, bm):
    nprog = N // bm
    kdim = w.shape[0]
    fdim = w.shape[1]
    nh = asd.shape[1] // 2
    if tdim == 0:
        x_spec = pl.BlockSpec((kdim, bm), lambda j: (0, j))
    else:
        x_spec = pl.BlockSpec((bm, kdim), lambda j: (j, 0))
    return pl.pallas_call(
        functools.partial(_proj_kernel, tdim=tdim),
        grid=(nprog,),
        in_specs=[
            x_spec,
            pl.BlockSpec((kdim, fdim), lambda j: (0, 0)),
            pl.BlockSpec((fdim, 2 * nh), lambda j: (0, 0)),
        ],
        out_specs=[
            pl.BlockSpec((bm, fdim), lambda j: (j, 0)),
            pl.BlockSpec((nh, bm), lambda j: (0, j)),
            pl.BlockSpec((bm, nh), lambda j: (j, 0)),
        ],
        out_shape=[
            jax.ShapeDtypeStruct((N, fdim), jnp.bfloat16),
            jax.ShapeDtypeStruct((nh, N), jnp.float32),
            jax.ShapeDtypeStruct((N, nh), jnp.float32),
        ],
    )(xin, w, asd)


def _attn1_proj_kernel(s_ref, d_ref, h_ref, b_ref, w2_ref, asd2_ref,
                       h2_ref, s2_ref, d2_ref, *, heads):
    outs = []
    for hd in range(heads):
        s_row = s_ref[hd:hd + 1, :]
        u1 = jnp.exp2(s_row).astype(jnp.bfloat16)
        u2 = jnp.exp2(0.2 * s_row).astype(jnp.bfloat16)
        smax = jnp.max(s_row)
        d_col = d_ref[:, hd:hd + 1]
        m = _lrelu(d_col + smax)
        v1 = jnp.exp2(d_col - m).astype(jnp.bfloat16)
        v2 = jnp.exp2(0.2 * d_col - m).astype(jnp.bfloat16)
        p = jnp.maximum(v1 * u1, v2 * u2)
        h_ext = jnp.concatenate(
            [h_ref[:, hd * HID:(hd + 1) * HID],
             jnp.ones((N, 1), jnp.bfloat16)], axis=1)
        pe = jax.lax.dot_general(
            p, h_ext, (((1,), (0,)), ((), ())),
            preferred_element_type=jnp.float32)
        outs.append(_elu(
            pe[:, :HID] / pe[:, HID:HID + 1]
            + b_ref[0:1, hd * HID:(hd + 1) * HID]).astype(jnp.bfloat16))
    out1 = jnp.concatenate(outs, axis=1)
    h2 = jax.lax.dot_general(
        out1, w2_ref[...].astype(jnp.bfloat16), (((1,), (0,)), ((), ())),
        preferred_element_type=jnp.float32)
    h2_ref[...] = h2.astype(jnp.bfloat16)
    sd = jax.lax.dot_general(
        h2, asd2_ref[...], (((1,), (0,)), ((), ())),
        preferred_element_type=jnp.float32)
    s2_ref[...] = sd[:, 0:1].T
    d2_ref[...] = sd[:, 1:2]


def _attn1_proj(s_r, d_t, h, b_r, w2, asd2, bi):
    ni = N // bi
    fdim = HEADS * HID
    return pl.pallas_call(
        functools.partial(_attn1_proj_kernel, heads=HEADS),
        grid=(ni,),
        in_specs=[
            pl.BlockSpec((HEADS, N), lambda it: (0, 0)),
            pl.BlockSpec((bi, HEADS), lambda it: (it, 0)),
            pl.BlockSpec((N, fdim), lambda it: (0, 0)),
            pl.BlockSpec((1, fdim), lambda it: (0, 0)),
            pl.BlockSpec((fdim, HID), lambda it: (0, 0)),
            pl.BlockSpec((HID, 2), lambda it: (0, 0)),
        ],
        out_specs=[
            pl.BlockSpec((bi, HID), lambda it: (it, 0)),
            pl.BlockSpec((1, bi), lambda it: (0, it)),
            pl.BlockSpec((bi, 1), lambda it: (it, 0)),
        ],
        out_shape=[
            jax.ShapeDtypeStruct((N, HID), jnp.bfloat16),
            jax.ShapeDtypeStruct((1, N), jnp.float32),
            jax.ShapeDtypeStruct((N, 1), jnp.float32),
        ],
        compiler_params=pltpu.CompilerParams(
            dimension_semantics=("parallel",)),
    )(s_r, d_t, h, b_r, w2, asd2)


def _attention(s_r, d_t, h, b_r, heads, bi, out_dtype):
    ni = N // bi
    fdim = heads * HID
    return pl.pallas_call(
        functools.partial(_attn_kernel, heads=heads),
        grid=(ni,),
        in_specs=[
            pl.BlockSpec((heads, N), lambda it: (0, 0)),
            pl.BlockSpec((bi, heads), lambda it: (it, 0)),
            pl.BlockSpec((N, fdim), lambda it: (0, 0)),
            pl.BlockSpec((1, fdim), lambda it: (0, 0)),
        ],
        out_specs=pl.BlockSpec((bi, fdim), lambda it: (it, 0)),
        out_shape=jax.ShapeDtypeStruct((N, fdim), out_dtype),
        compiler_params=pltpu.CompilerParams(
            dimension_semantics=("parallel",)),
    )(s_r, d_t, h, b_r)


def kernel(x, W1, a1_src, a1_dst, b1, W2, a2_src, a2_dst, b2):
    log2e = jnp.float32(1.4426950408889634)
    eye1 = jnp.eye(HEADS, dtype=jnp.float32)
    A1s = (eye1[:, None, :] * a1_src[:, :, None]).reshape(HEADS * HID, HEADS)
    A1d = (eye1[:, None, :] * a1_dst[:, :, None]).reshape(HEADS * HID, HEADS)
    asd1 = jnp.concatenate([A1s, A1d], axis=1) * log2e
    asd2 = jnp.concatenate([a2_src.T, a2_dst.T], axis=1) * log2e

    h1, s1, d1 = _projection(x, W1, asd1, tdim=0, bm=1024)
    h2, s2, d2 = _attn1_proj(s1, d1, h1, b1.reshape(1, HEADS * HID),
                             W2, asd2, bi=1024)
    out2 = _attention(s2, d2, h2, b2.reshape(1, HID), 1,
                      bi=1024, out_dtype=jnp.float32)
    return out2.reshape(1, N * HID)
